# Initial kernel scaffold; baseline (speedup 1.0000x reference)
import jax
import jax.numpy as jnp
from jax.experimental import pallas as pl


def kernel(ego, obs, lane, bound, embedding):
    raise NotImplementedError("write your pallas kernel here")


# speedup vs baseline: 9.6470x; 9.6470x over previous
import jax
import jax.numpy as jnp
from jax.experimental import pallas as pl

_DIM = 128
_OTHER_START = 1
_ROUTE_LEN = 1


def _make_body(seg_list, bb):
    def body(emb_ref, out_ref):
        e = emb_ref[...]
        for (lo, ln, t) in seg_list:
            out_ref[:, lo:lo + ln, :] = jnp.broadcast_to(
                e[t][None, None, :], (bb, ln, _DIM))
    return body


def kernel(ego, obs, lane, bound, embedding):
    batch = ego.shape[0]
    obs_len = obs.shape[1]
    lanes_len = lane.shape[1]
    bounds_len = bound.shape[1]
    total_len = 1 + obs_len + _ROUTE_LEN + lanes_len + bounds_len

    route_start = _OTHER_START + obs_len
    lanes_start = route_start + _ROUTE_LEN
    bounds_start = lanes_start + lanes_len
    segs = [
        (0, 1, 0),
        (_OTHER_START, obs_len, 2),
        (route_start, _ROUTE_LEN, 3),
        (lanes_start, lanes_len, 4),
        (bounds_start, bounds_len, 5),
    ]

    bb = 64
    while batch % bb != 0:
        bb //= 2
    grid = (batch // bb,)

    out = pl.pallas_call(
        _make_body(segs, bb),
        grid=grid,
        in_specs=[pl.BlockSpec((embedding.shape[0], _DIM), lambda i: (0, 0))],
        out_specs=pl.BlockSpec((bb, total_len, _DIM), lambda i: (i, 0, 0)),
        out_shape=jax.ShapeDtypeStruct((batch, total_len, _DIM),
                                       embedding.dtype),
    )(embedding)
    return out
